# baseline (speedup 1.0000x reference)
import jax
import jax.numpy as jnp
from jax import lax
from jax.experimental import pallas as pl
from jax.experimental.pallas import tpu as pltpu
from jax.experimental.pallas import tpu_sc as plsc

_N = 100000
_D = 128
_G = 2048
_TILE = 256
_CH = 128
_NFULL = _N // _TILE
_NT = _NFULL + 1
_NW = 32
_TPW = _NT // _NW
_EXTRA = _NT - _TPW * _NW
_MAXT = _TPW + 1
_NBUF = 3
_ACC_ROWS = _G + 8
_IDX_ROWS = 788


def _sc_body(x_hbm, idx_hbm, zc_hbm, out_hbm, idx_v, rows_v, acc, gsem):
    c = lax.axis_index("c")
    s = lax.axis_index("s")
    wid = c * 16 + s

    pltpu.sync_copy(zc_hbm, acc.at[pl.ds(s * 128, 128)])

    @pl.when(s == 0)
    def _():
        pltpu.sync_copy(zc_hbm.at[pl.ds(0, 8)], acc.at[pl.ds(_G, 8)])

    ntiles = jnp.where(wid < _EXTRA, _MAXT, _TPW)
    start = wid * _TPW + jnp.minimum(wid, _EXTRA)

    astart = ((2 * start) // 8) * 8
    off = 2 * start - astart
    pltpu.sync_copy(idx_hbm.at[pl.ds(astart, 32)], idx_v)

    plsc.subcore_barrier()

    def issue(k, b):
        t = start + k
        xbase = jnp.where(t == _NT - 1, _N - _TILE, t * _TILE)
        pltpu.async_copy(
            x_hbm.at[pl.ds(xbase, _TILE)], rows_v.at[b], gsem.at[b]
        )

    def wait_b(b):
        pltpu.make_async_copy(
            x_hbm.at[pl.ds(0, _TILE)], rows_v.at[b], gsem.at[b]
        ).wait()

    for b in range(_NBUF):
        issue(b, b)

    for k in range(_MAXT):
        b = k % _NBUF

        @pl.when(k < ntiles)
        def _(k=k, b=b):
            wait_b(b)
            pltpu.sync_copy(
                rows_v.at[b, pl.ds(0, _CH)],
                acc.at[idx_v.at[off + 2 * k]],
                add=True,
            )
            pltpu.sync_copy(
                rows_v.at[b, pl.ds(_CH, _CH)],
                acc.at[idx_v.at[off + 2 * k + 1]],
                add=True,
            )

        @pl.when(k + _NBUF < ntiles)
        def _(k=k, b=b):
            issue(k + _NBUF, b)

    plsc.subcore_barrier()
    pltpu.sync_copy(acc.at[pl.ds(s * 128, 128)], out_hbm.at[c, pl.ds(s * 128, 128)])


_sc_call = pl.kernel(
    _sc_body,
    out_type=jax.ShapeDtypeStruct((2, _G, _D), jnp.float32),
    mesh=plsc.VectorSubcoreMesh(core_axis_name="c", subcore_axis_name="s"),
    scratch_types=[
        pltpu.VMEM((32, _CH), jnp.int32),
        pltpu.VMEM((_NBUF, _TILE, _D), jnp.float32),
        pltpu.VMEM_SHARED((_ACC_ROWS, _D), jnp.float32),
        pltpu.SemaphoreType.DMA((_NBUF,)),
    ],
)


def _combine_body(p_ref, o_ref):
    o_ref[...] = p_ref[0] + p_ref[1]


_combine = pl.pallas_call(
    _combine_body,
    out_shape=jax.ShapeDtypeStruct((_G, _D), jnp.float32),
    grid=(8,),
    in_specs=[pl.BlockSpec((2, _G // 8, _D), lambda i: (0, i, 0))],
    out_specs=pl.BlockSpec((_G // 8, _D), lambda i: (i, 0)),
)


def kernel(x, edge_index, edge_attr, batch):
    b = batch.astype(jnp.int32)
    main = b[: _NFULL * _TILE].reshape(_NFULL * 2, _CH)
    tail = jnp.concatenate(
        [jnp.full((_TILE - (_N - _NFULL * _TILE),), _G, jnp.int32), b[_NFULL * _TILE :]]
    ).reshape(2, _CH)
    pad = jnp.full((_IDX_ROWS - _NFULL * 2 - 2, _CH), _G, jnp.int32)
    idxs = jnp.concatenate([main, tail, pad], 0)
    zc = jnp.zeros((_CH, _D), jnp.float32)
    partials = _sc_call(x, idxs, zc)
    return _combine(partials)


# speedup vs baseline: 1.0321x; 1.0321x over previous
import jax
import jax.numpy as jnp
from jax import lax
from jax.experimental import pallas as pl
from jax.experimental.pallas import tpu as pltpu
from jax.experimental.pallas import tpu_sc as plsc

_N = 100000
_D = 128
_G = 2048
_TILE = 128
_NFULL = _N // _TILE
_REM = _N - _NFULL * _TILE
_NT = _NFULL + 1
_NW = 32
_TPW = _NT // _NW
_EXTRA = _NT - _TPW * _NW
_MAXT = _TPW + 1
_NBUF = 5
_ACC_ROWS = _G + 8


def _sc_body(x_hbm, idx_hbm, zc_hbm, out_hbm, idx_v, rows_v, acc, gsem):
    c = lax.axis_index("c")
    s = lax.axis_index("s")
    wid = c * 16 + s

    pltpu.sync_copy(zc_hbm, acc.at[pl.ds(s * 128, 128)])

    @pl.when(s == 0)
    def _():
        pltpu.sync_copy(zc_hbm.at[pl.ds(0, 8)], acc.at[pl.ds(_G, 8)])

    ntiles = jnp.where(wid < _EXTRA, _MAXT, _TPW)
    start = wid * _TPW + jnp.minimum(wid, _EXTRA)

    astart = (start // 8) * 8
    off = start - astart
    pltpu.sync_copy(idx_hbm.at[pl.ds(astart, 32)], idx_v)

    dummyv = jnp.full((16,), _G, jnp.int32)

    @pl.when(ntiles == _TPW)
    def _():
        r = off + _TPW
        for j in range(8):
            idx_v[r, pl.ds(j * 16, 16)] = dummyv

    plsc.subcore_barrier()

    def xbase_of(t):
        return jnp.where(
            t < _NFULL, t * _TILE, jnp.where(t == _NFULL, _N - _TILE, 0)
        )

    def issue(t, b):
        pltpu.async_copy(
            x_hbm.at[pl.ds(xbase_of(t), _TILE)], rows_v.at[b], gsem.at[b]
        )

    def wait_b(b):
        pltpu.make_async_copy(
            x_hbm.at[pl.ds(0, _TILE)], rows_v.at[b], gsem.at[b]
        ).wait()

    for b in range(_NBUF):
        issue(start + b, b)

    def outer(i, carry):
        kb = i * _NBUF
        for b in range(_NBUF):
            k = kb + b
            wait_b(b)
            pltpu.sync_copy(rows_v.at[b], acc.at[idx_v.at[off + k]], add=True)

            @pl.when(k + _NBUF < _MAXT)
            def _(k=k, b=b):
                issue(start + k + _NBUF, b)

        return carry

    lax.fori_loop(0, _MAXT // _NBUF, outer, 0)

    plsc.subcore_barrier()
    pltpu.sync_copy(acc.at[pl.ds(s * 128, 128)], out_hbm.at[c, pl.ds(s * 128, 128)])


_sc_call = pl.kernel(
    _sc_body,
    out_type=jax.ShapeDtypeStruct((2, _G, _D), jnp.float32),
    mesh=plsc.VectorSubcoreMesh(core_axis_name="c", subcore_axis_name="s"),
    scratch_types=[
        pltpu.VMEM((32, _TILE), jnp.int32),
        pltpu.VMEM((_NBUF, _TILE, _D), jnp.float32),
        pltpu.VMEM_SHARED((_ACC_ROWS, _D), jnp.float32),
        pltpu.SemaphoreType.DMA((_NBUF,)),
    ],
)


def _combine_body(p_ref, o_ref):
    o_ref[...] = p_ref[0] + p_ref[1]


_combine = pl.pallas_call(
    _combine_body,
    out_shape=jax.ShapeDtypeStruct((_G, _D), jnp.float32),
    grid=(8,),
    in_specs=[pl.BlockSpec((2, _G // 8, _D), lambda i: (0, i, 0))],
    out_specs=pl.BlockSpec((_G // 8, _D), lambda i: (i, 0)),
)


def kernel(x, edge_index, edge_attr, batch):
    b = batch.astype(jnp.int32)
    main = b[: _NFULL * _TILE].reshape(_NFULL, _TILE)
    tail = jnp.concatenate(
        [jnp.full((_TILE - _REM,), _G, jnp.int32), b[_NFULL * _TILE :]]
    ).reshape(1, _TILE)
    pad = jnp.full((2, _TILE), _G, jnp.int32)
    idxs = jnp.concatenate([main, tail, pad], 0)
    zc = jnp.zeros((_TILE, _D), jnp.float32)
    partials = _sc_call(x, idxs, zc)
    return _combine(partials)


# speedup vs baseline: 1.1114x; 1.0769x over previous
import jax
import jax.numpy as jnp
from jax import lax
from jax.experimental import pallas as pl
from jax.experimental.pallas import tpu as pltpu
from jax.experimental.pallas import tpu_sc as plsc

_N = 100000
_D = 128
_G = 2048
_TILE = 128
_NFULL = _N // _TILE
_REM = _N - _NFULL * _TILE
_NT = _NFULL + 1
_NW = 32
_TPW = _NT // _NW
_EXTRA = _NT - _TPW * _NW
_MAXT = _TPW + 1
_NBUF = 5
_ACC_ROWS = _G + 8


def _sc_body(x_hbm, idx_hbm, out_hbm, idx_v, rows_v, zbuf, acc, gsem, ssem):
    c = lax.axis_index("c")
    s = lax.axis_index("s")
    wid = c * 16 + s

    zv = jnp.zeros((16,), jnp.float32)

    def zrow(i, carry):
        for j in range(8):
            zbuf[i, pl.ds(j * 16, 16)] = zv
        return carry

    lax.fori_loop(0, _TILE, zrow, 0)
    pltpu.sync_copy(zbuf, acc.at[pl.ds(s * 128, 128)])

    @pl.when(s == 0)
    def _():
        pltpu.sync_copy(zbuf.at[pl.ds(0, 8)], acc.at[pl.ds(_G, 8)])

    ntiles = jnp.where(wid < _EXTRA, _MAXT, _TPW)
    start = wid * _TPW + jnp.minimum(wid, _EXTRA)

    astart = (start // 8) * 8
    off = start - astart
    pltpu.sync_copy(idx_hbm.at[pl.ds(astart, 32)], idx_v)

    dummyv = jnp.full((16,), _G, jnp.int32)

    @pl.when(ntiles == _TPW)
    def _():
        r = off + _TPW
        for j in range(8):
            idx_v[r, pl.ds(j * 16, 16)] = dummyv

    plsc.subcore_barrier()

    def xbase_of(t):
        return jnp.where(
            t < _NFULL, t * _TILE, jnp.where(t == _NFULL, _N - _TILE, 0)
        )

    def issue(t, b):
        pltpu.async_copy(
            x_hbm.at[pl.ds(xbase_of(t), _TILE)], rows_v.at[b], gsem.at[b]
        )

    def wait_b(b):
        pltpu.make_async_copy(
            x_hbm.at[pl.ds(0, _TILE)], rows_v.at[b], gsem.at[b]
        ).wait()

    def scat(k, b):
        pltpu.async_copy(
            rows_v.at[b], acc.at[idx_v.at[off + k]], ssem.at[b], add=True
        )

    def wait_s(b):
        pltpu.make_async_copy(
            rows_v.at[b], acc.at[idx_v.at[0]], ssem.at[b]
        ).wait()

    issue(start, 0)
    issue(start + 1, 1)
    for k in range(_MAXT):
        b = k % _NBUF
        bg = (k + 2) % _NBUF
        if 3 <= k:
            wait_s(bg)
        if k + 2 < _MAXT:
            issue(start + k + 2, bg)
        wait_b(b)
        scat(k, b)
    for k in range(_MAXT - 3, _MAXT):
        wait_s(k % _NBUF)

    plsc.subcore_barrier()
    pltpu.sync_copy(acc.at[pl.ds(s * 128, 128)], out_hbm.at[c, pl.ds(s * 128, 128)])


_sc_call = pl.kernel(
    _sc_body,
    out_type=jax.ShapeDtypeStruct((2, _G, _D), jnp.float32),
    mesh=plsc.VectorSubcoreMesh(core_axis_name="c", subcore_axis_name="s"),
    scratch_types=[
        pltpu.VMEM((32, _TILE), jnp.int32),
        pltpu.VMEM((_NBUF, _TILE, _D), jnp.float32),
        pltpu.VMEM((_TILE, _D), jnp.float32),
        pltpu.VMEM_SHARED((_ACC_ROWS, _D), jnp.float32),
        pltpu.SemaphoreType.DMA((_NBUF,)),
        pltpu.SemaphoreType.DMA((_NBUF,)),
    ],
)


def _combine_body(p_ref, o_ref):
    o_ref[...] = p_ref[0] + p_ref[1]


_combine = pl.pallas_call(
    _combine_body,
    out_shape=jax.ShapeDtypeStruct((_G, _D), jnp.float32),
    grid=(8,),
    in_specs=[pl.BlockSpec((2, _G // 8, _D), lambda i: (0, i, 0))],
    out_specs=pl.BlockSpec((_G // 8, _D), lambda i: (i, 0)),
)


def kernel(x, edge_index, edge_attr, batch):
    b = batch.astype(jnp.int32)
    main = b[: _NFULL * _TILE].reshape(_NFULL, _TILE)
    tail = jnp.concatenate(
        [jnp.full((_TILE - _REM,), _G, jnp.int32), b[_NFULL * _TILE :]]
    ).reshape(1, _TILE)
    pad = jnp.full((2, _TILE), _G, jnp.int32)
    idxs = jnp.concatenate([main, tail, pad], 0)
    partials = _sc_call(x, idxs)
    return _combine(partials)


# speedup vs baseline: 1.1285x; 1.0154x over previous
import jax
import jax.numpy as jnp
from jax import lax
from jax.experimental import pallas as pl
from jax.experimental.pallas import tpu as pltpu
from jax.experimental.pallas import tpu_sc as plsc

_N = 100000
_D = 128
_G = 2048
_TILE = 128
_NFULL = _N // _TILE
_REM = _N - _NFULL * _TILE
_NT = _NFULL + 1
_NW = 32
_TPW = _NT // _NW
_EXTRA = _NT - _TPW * _NW
_MAXT = _TPW + 1
_NBUF = 5
_ACC_ROWS = _G + 8


def _sc_body(x_hbm, idx_hbm, out_hbm, idx_v, rows_v, zbuf, acc, gsem, ssem):
    c = lax.axis_index("c")
    s = lax.axis_index("s")
    wid = c * 16 + s

    ntiles = jnp.where(wid < _EXTRA, _MAXT, _TPW)
    start = wid * _TPW + jnp.minimum(wid, _EXTRA)
    astart = (start // 8) * 8
    off = start - astart

    def xbase_of(t):
        return jnp.where(
            t < _NFULL, t * _TILE, jnp.where(t == _NFULL, _N - _TILE, 0)
        )

    def issue(t, b):
        pltpu.async_copy(
            x_hbm.at[pl.ds(xbase_of(t), _TILE)], rows_v.at[b], gsem.at[b]
        )

    def wait_b(b):
        pltpu.make_async_copy(
            x_hbm.at[pl.ds(0, _TILE)], rows_v.at[b], gsem.at[b]
        ).wait()

    issue(start, 0)
    issue(start + 1, 1)
    pltpu.sync_copy(idx_hbm.at[pl.ds(astart, 32)], idx_v)

    zv = jnp.zeros((16,), jnp.float32)

    def zrow(i, carry):
        for j in range(8):
            zbuf[i, pl.ds(j * 16, 16)] = zv
        return carry

    lax.fori_loop(0, _TILE, zrow, 0)
    pltpu.sync_copy(zbuf, acc.at[pl.ds(s * 128, 128)])

    @pl.when(s == 0)
    def _():
        pltpu.sync_copy(zbuf.at[pl.ds(0, 8)], acc.at[pl.ds(_G, 8)])

    dummyv = jnp.full((16,), _G, jnp.int32)

    @pl.when(ntiles == _TPW)
    def _():
        r = off + _TPW
        for j in range(8):
            idx_v[r, pl.ds(j * 16, 16)] = dummyv

    plsc.subcore_barrier()

    def scat(k, b):
        pltpu.async_copy(
            rows_v.at[b], acc.at[idx_v.at[off + k]], ssem.at[b], add=True
        )

    def wait_s(b):
        pltpu.make_async_copy(
            rows_v.at[b], acc.at[idx_v.at[0]], ssem.at[b]
        ).wait()

    for k in range(_MAXT):
        b = k % _NBUF
        bg = (k + 2) % _NBUF
        if 3 <= k:
            wait_s(bg)
        if k + 2 < _MAXT:
            issue(start + k + 2, bg)
        wait_b(b)
        scat(k, b)
    for k in range(_MAXT - 3, _MAXT):
        wait_s(k % _NBUF)

    plsc.subcore_barrier()
    pltpu.sync_copy(acc.at[pl.ds(s * 128, 128)], out_hbm.at[c, pl.ds(s * 128, 128)])


_sc_call = pl.kernel(
    _sc_body,
    out_type=jax.ShapeDtypeStruct((2, _G, _D), jnp.float32),
    mesh=plsc.VectorSubcoreMesh(core_axis_name="c", subcore_axis_name="s"),
    scratch_types=[
        pltpu.VMEM((32, _TILE), jnp.int32),
        pltpu.VMEM((_NBUF, _TILE, _D), jnp.float32),
        pltpu.VMEM((_TILE, _D), jnp.float32),
        pltpu.VMEM_SHARED((_ACC_ROWS, _D), jnp.float32),
        pltpu.SemaphoreType.DMA((_NBUF,)),
        pltpu.SemaphoreType.DMA((_NBUF,)),
    ],
)


def _combine_body(p_ref, o_ref):
    o_ref[...] = p_ref[0] + p_ref[1]


_combine = pl.pallas_call(
    _combine_body,
    out_shape=jax.ShapeDtypeStruct((_G, _D), jnp.float32),
    grid=(8,),
    in_specs=[pl.BlockSpec((2, _G // 8, _D), lambda i: (0, i, 0))],
    out_specs=pl.BlockSpec((_G // 8, _D), lambda i: (i, 0)),
)


def kernel(x, edge_index, edge_attr, batch):
    b = batch.astype(jnp.int32)
    main = b[: _NFULL * _TILE].reshape(_NFULL, _TILE)
    tail = jnp.concatenate(
        [jnp.full((_TILE - _REM,), _G, jnp.int32), b[_NFULL * _TILE :]]
    ).reshape(1, _TILE)
    pad = jnp.full((2, _TILE), _G, jnp.int32)
    idxs = jnp.concatenate([main, tail, pad], 0)
    partials = _sc_call(x, idxs)
    return _combine(partials)


# speedup vs baseline: 1.1319x; 1.0030x over previous
import jax
import jax.numpy as jnp
from jax import lax
from jax.experimental import pallas as pl
from jax.experimental.pallas import tpu as pltpu
from jax.experimental.pallas import tpu_sc as plsc

_N = 100000
_D = 128
_G = 2048
_TILE = 128
_NFULL = _N // _TILE
_REM = _N - _NFULL * _TILE
_NT = _NFULL + 1
_NW = 32
_TPW = _NT // _NW
_EXTRA = _NT - _TPW * _NW
_MAXT = _TPW + 1
_NBUF = 5
_LEAD = 2
_ACC_ROWS = _G + 8


def _sc_body(x_hbm, idx_hbm, out_hbm, idx_v, rows_v, zbuf, acc, gsem, ssem, zsem):
    c = lax.axis_index("c")
    s = lax.axis_index("s")
    wid = c * 16 + s

    ntiles = jnp.where(wid < _EXTRA, _MAXT, _TPW)
    start = wid * _TPW + jnp.minimum(wid, _EXTRA)
    astart = (start // 8) * 8
    off = start - astart

    def xbase_of(t):
        return jnp.where(
            t < _NFULL, t * _TILE, jnp.where(t == _NFULL, _N - _TILE, 0)
        )

    def issue(t, b):
        pltpu.async_copy(
            x_hbm.at[pl.ds(xbase_of(t), _TILE)], rows_v.at[b], gsem.at[b]
        )

    def wait_b(b):
        pltpu.make_async_copy(
            x_hbm.at[pl.ds(0, _TILE)], rows_v.at[b], gsem.at[b]
        ).wait()

    for b in range(_LEAD):
        issue(start + b, b)
    pltpu.sync_copy(idx_hbm.at[pl.ds(astart, 32)], idx_v)

    zv = jnp.zeros((16,), jnp.float32)
    for i in range(8):
        for j in range(8):
            zbuf[i, pl.ds(j * 16, 16)] = zv
    for j in range(16):
        pltpu.async_copy(zbuf, acc.at[pl.ds(s * 128 + 8 * j, 8)], zsem)

    @pl.when(s == 0)
    def _():
        pltpu.async_copy(zbuf, acc.at[pl.ds(_G, 8)], zsem)

    for j in range(16):
        pltpu.make_async_copy(zbuf, acc.at[pl.ds(0, 8)], zsem).wait()

    @pl.when(s == 0)
    def _():
        pltpu.make_async_copy(zbuf, acc.at[pl.ds(0, 8)], zsem).wait()

    dummyv = jnp.full((16,), _G, jnp.int32)

    @pl.when(ntiles == _TPW)
    def _():
        r = off + _TPW
        for j in range(8):
            idx_v[r, pl.ds(j * 16, 16)] = dummyv

    plsc.subcore_barrier()

    def scat(k, b):
        pltpu.async_copy(
            rows_v.at[b], acc.at[idx_v.at[off + k]], ssem.at[b], add=True
        )

    def wait_s(b):
        pltpu.make_async_copy(
            rows_v.at[b], acc.at[idx_v.at[0]], ssem.at[b]
        ).wait()

    _LAG = _NBUF - _LEAD
    for k in range(_MAXT):
        b = k % _NBUF
        bg = (k + _LEAD) % _NBUF
        if _LAG <= k:
            wait_s(bg)
        if k + _LEAD < _MAXT:
            issue(start + k + _LEAD, bg)
        wait_b(b)
        scat(k, b)
    for k in range(_MAXT - _LAG, _MAXT):
        wait_s(k % _NBUF)

    plsc.subcore_barrier()
    pltpu.sync_copy(acc.at[pl.ds(s * 128, 128)], out_hbm.at[c, pl.ds(s * 128, 128)])


_sc_call = pl.kernel(
    _sc_body,
    out_type=jax.ShapeDtypeStruct((2, _G, _D), jnp.float32),
    mesh=plsc.VectorSubcoreMesh(core_axis_name="c", subcore_axis_name="s"),
    scratch_types=[
        pltpu.VMEM((32, _TILE), jnp.int32),
        pltpu.VMEM((_NBUF, _TILE, _D), jnp.float32),
        pltpu.VMEM((8, _D), jnp.float32),
        pltpu.VMEM_SHARED((_ACC_ROWS, _D), jnp.float32),
        pltpu.SemaphoreType.DMA((_NBUF,)),
        pltpu.SemaphoreType.DMA((_NBUF,)),
        pltpu.SemaphoreType.DMA,
    ],
)


def _combine_body(p_ref, o_ref):
    o_ref[...] = p_ref[0] + p_ref[1]


_combine = pl.pallas_call(
    _combine_body,
    out_shape=jax.ShapeDtypeStruct((_G, _D), jnp.float32),
    grid=(8,),
    in_specs=[pl.BlockSpec((2, _G // 8, _D), lambda i: (0, i, 0))],
    out_specs=pl.BlockSpec((_G // 8, _D), lambda i: (i, 0)),
)


def kernel(x, edge_index, edge_attr, batch):
    b = batch.astype(jnp.int32)
    main = b[: _NFULL * _TILE].reshape(_NFULL, _TILE)
    tail = jnp.concatenate(
        [jnp.full((_TILE - _REM,), _G, jnp.int32), b[_NFULL * _TILE :]]
    ).reshape(1, _TILE)
    pad = jnp.full((2, _TILE), _G, jnp.int32)
    idxs = jnp.concatenate([main, tail, pad], 0)
    partials = _sc_call(x, idxs)
    return _combine(partials)
